# Initial kernel scaffold; baseline (speedup 1.0000x reference)
import jax
import jax.numpy as jnp
from jax.experimental import pallas as pl


def kernel(x_embed, prompt_old, prompt_new, prompt_key_old, prompt_key_new):
    raise NotImplementedError("write your pallas kernel here")


# speedup vs baseline: 1.3599x; 1.3599x over previous
import functools

import jax
import jax.numpy as jnp
from jax import lax
from jax.experimental import pallas as pl
from jax.experimental.pallas import tpu as pltpu
from jax.experimental.pallas import tpu_sc as plsc

_B = 512
_T = 128
_D = 1024
_PP = 4096
_P = 2 * _PP
_L = 5
_K = 8
_OUT_ROWS = _K * _L + _T

_NC, _NS = 2, 16
_NW = _NC * _NS
_BPW = _B // _NW
_PAIRS = _BPW // 2


def _mean_norm_body(x_ref, q_ref):
    m = jnp.mean(x_ref[...], axis=1)
    ss = jnp.sum(m * m, axis=1, keepdims=True)
    q_ref[...] = m * lax.rsqrt(jnp.maximum(ss, 1e-12))


def _sim_body(q_ref, k_ref, s_ref):
    k = k_ref[...]
    ss = jnp.sum(k * k, axis=1, keepdims=True)
    kn = k * lax.rsqrt(jnp.maximum(ss, 1e-12))
    s_ref[...] = lax.dot_general(
        q_ref[...], kn, (((1,), (1,)), ((), ())),
        preferred_element_type=jnp.float32)


def _topk_body(s_ref, idx_ref, row_ref, acc_ref):
    val = s_ref[...]
    it = lax.broadcasted_iota(jnp.int32, val.shape, 1)
    cols = []
    tot = jnp.float32(0.0)
    for _ in range(_K):
        m = jnp.max(val, axis=1, keepdims=True)
        sel = jnp.min(jnp.where(val == m, it, _P), axis=1, keepdims=True)
        cols.append(sel)
        tot = tot + jnp.sum(m)
        val = jnp.where(it == sel, -jnp.inf, val)
    idx_ref[...] = jnp.concatenate(cols, axis=1)
    row_ref[...] = jnp.concatenate(
        [cols[k] * _L + j for k in range(_K) for j in range(_L)], axis=1)
    acc_ref[...] = jnp.full((1, 1, 128), tot, jnp.float32)


def _sc_body(prompt_hbm, row_hbm, x_hbm, out_hbm, rowidx, rows, xbuf, sem):
    wid = lax.axis_index("s") * _NC + lax.axis_index("c")

    def pair_body(i, carry):
        b0 = wid * _BPW + i * 2
        pltpu.sync_copy(row_hbm.at[pl.ds(b0 * _K * _L, 2 * _K * _L)], rowidx)
        pltpu.async_copy(prompt_hbm.at[rowidx], rows, sem).wait()
        pltpu.sync_copy(rows.at[pl.ds(0, _K * _L)],
                        out_hbm.at[pl.ds(b0 * _OUT_ROWS, _K * _L)])
        pltpu.sync_copy(rows.at[pl.ds(_K * _L, _K * _L)],
                        out_hbm.at[pl.ds((b0 + 1) * _OUT_ROWS, _K * _L)])
        return carry

    lax.fori_loop(0, _PAIRS, pair_body, 0)

    def copy_body(i, carry):
        b = wid * _BPW + lax.div(i, 4)
        c4 = i - lax.div(i, 4) * 4
        pltpu.sync_copy(x_hbm.at[pl.ds(b * _T + c4 * 32, 32)], xbuf)
        pltpu.sync_copy(xbuf,
                        out_hbm.at[pl.ds(b * _OUT_ROWS + _K * _L + c4 * 32, 32)])
        return carry

    lax.fori_loop(0, _BPW * 4, copy_body, 0)


@functools.cache
def _sc_gather():
    return pl.kernel(
        _sc_body,
        out_type=jax.ShapeDtypeStruct((_B * _OUT_ROWS, _D), jnp.float32),
        mesh=plsc.VectorSubcoreMesh(core_axis_name="c", subcore_axis_name="s",
                                    num_cores=_NC, num_subcores=_NS),
        scratch_types=[
            pltpu.VMEM((2 * _K * _L,), jnp.int32),
            pltpu.VMEM((2 * _K * _L, _D), jnp.float32),
            pltpu.VMEM((32, _D), jnp.float32),
            pltpu.SemaphoreType.DMA,
        ],
    )


@jax.jit
def kernel(x_embed, prompt_old, prompt_new, prompt_key_old, prompt_key_new):
    keys = jnp.concatenate([prompt_key_old, prompt_key_new], axis=0)
    prompt2d = jnp.concatenate([prompt_old, prompt_new], axis=0)
    prompt2d = prompt2d.reshape(_P * _L, _D)

    q_norm = pl.pallas_call(
        _mean_norm_body,
        grid=(16,),
        in_specs=[pl.BlockSpec((_B // 16, _T, _D), lambda i: (i, 0, 0))],
        out_specs=pl.BlockSpec((_B // 16, _D), lambda i: (i, 0)),
        out_shape=jax.ShapeDtypeStruct((_B, _D), jnp.float32),
    )(x_embed)

    sim = pl.pallas_call(
        _sim_body,
        grid=(8,),
        in_specs=[
            pl.BlockSpec((_B, _D), lambda i: (0, 0)),
            pl.BlockSpec((_P // 8, _D), lambda i: (i, 0)),
        ],
        out_specs=pl.BlockSpec((_B, _P // 8), lambda i: (0, i)),
        out_shape=jax.ShapeDtypeStruct((_B, _P), jnp.float32),
    )(q_norm, keys)

    idx, row, acc = pl.pallas_call(
        _topk_body,
        grid=(4,),
        in_specs=[pl.BlockSpec((_B // 4, _P), lambda i: (i, 0))],
        out_specs=[
            pl.BlockSpec((_B // 4, _K), lambda i: (i, 0)),
            pl.BlockSpec((_B // 4, _K * _L), lambda i: (i, 0)),
            pl.BlockSpec((1, 1, 128), lambda i: (i, 0, 0)),
        ],
        out_shape=[
            jax.ShapeDtypeStruct((_B, _K), jnp.int32),
            jax.ShapeDtypeStruct((_B, _K * _L), jnp.int32),
            jax.ShapeDtypeStruct((4, 1, 128), jnp.float32),
        ],
    )(sim)

    out2d = _sc_gather()(prompt2d, row.reshape(-1),
                         x_embed.reshape(_B * _T, _D))
    prompted = out2d.reshape(_B, _OUT_ROWS, _D)
    reduce_sim = jnp.sum(acc[:, 0, 0]) / _B
    return prompted, reduce_sim, sim, idx
